# baseline (speedup 1.0000x reference)
import functools

import jax
import jax.numpy as jnp
from jax import lax
from jax.experimental import pallas as pl
from jax.experimental.pallas import tpu as pltpu
from jax.experimental.pallas import tpu_sc as plsc

_N = 50000
_DIM = 16
_E = 1600000
_NC = 2
_NS = 16
_LANES = 128
_BIGC = 8
_NBANK = 4
_RPW = 784
_R = _NS * _RPW
_EP = _R * _LANES
_NBIG = _RPW // _BIGC
_NP = 50048
_NPT = _NP // _NS


def _prep_body(x_ref, g_ref, e_ref, r_ref, h_ref):
    xb = jnp.dot(x_ref[...], e_ref[...], preferred_element_type=jnp.float32)
    prod = xb * g_ref[...]
    h = jnp.dot(prod, r_ref[...], preferred_element_type=jnp.float32)
    nrm = jnp.maximum(jnp.sqrt(jnp.sum(h * h, axis=1, keepdims=True)), 1e-12)
    h_ref[...] = h / nrm


def _prep(x, gauges):
    B = 2000
    g2 = gauges.reshape(_N, _DIM * _DIM)
    em = jnp.kron(jnp.eye(_DIM, dtype=jnp.float32),
                  jnp.ones((1, _DIM), dtype=jnp.float32))
    rm = jnp.kron(jnp.ones((_DIM, 1), dtype=jnp.float32),
                  jnp.eye(_DIM, dtype=jnp.float32))
    return pl.pallas_call(
        _prep_body,
        grid=(_N // B,),
        in_specs=[
            pl.BlockSpec((B, _DIM), lambda i: (i, 0)),
            pl.BlockSpec((B, _DIM * _DIM), lambda i: (i, 0)),
            pl.BlockSpec(em.shape, lambda i: (0, 0)),
            pl.BlockSpec(rm.shape, lambda i: (0, 0)),
        ],
        out_specs=pl.BlockSpec((B, _DIM), lambda i: (i, 0)),
        out_shape=jax.ShapeDtypeStruct((_N, _DIM), jnp.float32),
    )(x, g2, em, rm)


def _msgpass_body(table_h, ed_h, out1_h, out2_h,
                  acc_a, acc_b, edata, rm, gsems, ssems, isem):
    cc = lax.axis_index("c")
    s = lax.axis_index("s")
    nbase = s * _NPT
    row0 = s * _RPW

    def _zrow(i, _):
        rm[0, i] = jnp.zeros((_DIM,), jnp.float32)
        return 0

    def _zero_acc(acc):
        def _zcp(i, _):
            pltpu.sync_copy(rm.at[0, pl.ds(0, _LANES)],
                            acc.at[pl.ds(nbase + i * _LANES, _LANES)])
            return 0

        nfull = _NPT // _LANES
        lax.fori_loop(0, nfull, _zcp, 0)
        rem = _NPT - nfull * _LANES
        pltpu.sync_copy(rm.at[0, pl.ds(0, rem)],
                        acc.at[pl.ds(nbase + nfull * _LANES, rem)])

    def _fire_idx(ib, r0):
        pltpu.async_copy(ed_h.at[pl.ds(r0, _BIGC)], edata.at[ib], isem)

    def _drain_idx(ib, goff):
        pltpu.make_async_copy(ed_h.at[pl.ds(0, _BIGC)], edata.at[ib],
                              isem).wait()
        if goff is not None:
            for r in range(_BIGC):
                for j in range(_LANES // _DIM):
                    sl = pl.ds(j * _DIM, _DIM)
                    edata[ib, r, 0, sl] = edata[ib, r, 0, sl] + goff

    def _drain_gather(m):
        pltpu.make_async_copy(table_h.at[pl.ds(0, _LANES)],
                              rm.at[m, pl.ds(0, _LANES)], gsems[m]).wait()

    def _pipeline(phase):
        goff = None if phase == 1 else cc * _NP

        def _fire_gather(ib, k, m):
            tbl = table_h if phase == 1 else out1_h
            pltpu.async_copy(tbl.at[edata.at[ib, k, 0]],
                             rm.at[m, pl.ds(0, _LANES)], gsems[m])

        def _drain_scatters(m):
            if phase == 1:
                pltpu.make_async_copy(table_h.at[pl.ds(0, _LANES)],
                                      rm.at[m, pl.ds(0, _LANES)],
                                      ssems[m]).wait()
            else:
                pltpu.make_async_copy(table_h.at[pl.ds(0, 2 * _LANES)],
                                      rm.at[m], ssems[m]).wait()

        def _compute(ib, k, m):
            if phase == 1:
                def _blk(b, _):
                    k0 = plsc.bitcast(
                        edata[ib, k, 2, pl.ds(b * _DIM, _DIM)], jnp.float32)
                    k1 = plsc.bitcast(
                        edata[ib, k, 3, pl.ds(b * _DIM, _DIM)], jnp.float32)
                    kv = jnp.where(cc == 0, k0, k1)
                    for ee in range(_DIM):
                        e = b * _DIM + ee
                        rm[m, e] = rm[m, e] * kv[ee]
                    return 0

                lax.fori_loop(0, _LANES // _DIM, _blk, 0)
                pltpu.async_copy(rm.at[m, pl.ds(0, _LANES)],
                                 acc_a.at[edata.at[ib, k, 1]], ssems[m],
                                 add=True)
            else:
                def _blk(b, _):
                    k0 = plsc.bitcast(
                        edata[ib, k, 2, pl.ds(b * _DIM, _DIM)], jnp.float32)
                    k1 = plsc.bitcast(
                        edata[ib, k, 3, pl.ds(b * _DIM, _DIM)], jnp.float32)
                    for ee in range(_DIM):
                        e = b * _DIM + ee
                        v = rm[m, e]
                        rm[m, _LANES + e] = v * k1[ee]
                        rm[m, e] = v * k0[ee]
                    return 0

                lax.fori_loop(0, _LANES // _DIM, _blk, 0)
                pltpu.async_copy(rm.at[m, pl.ds(0, _LANES)],
                                 acc_a.at[edata.at[ib, k, 1]], ssems[m],
                                 add=True)
                pltpu.async_copy(rm.at[m, pl.ds(_LANES, _LANES)],
                                 acc_b.at[edata.at[ib, k, 1]], ssems[m],
                                 add=True)

        def _chunk(ib, k, B):
            static = isinstance(B, int)
            m = k % _NBANK
            nx = (k + 1) % _NBANK
            if not (static and B == 0 and k < 3):
                _drain_scatters(nx)
            if k == 3 and not (static and B == 0):
                if static:
                    if B < _NBIG - 1:
                        _fire_idx(1 - ib, row0 + (B + 1) * _BIGC)
                else:
                    @pl.when(B < _NBIG - 1)
                    def _():
                        _fire_idx(1 - ib, row0 + (B + 1) * _BIGC)
            if k < _BIGC - 1:
                _fire_gather(ib, k + 1, nx)
            else:
                if static:
                    if B < _NBIG - 1:
                        _drain_idx(1 - ib, goff)
                        _fire_gather(1 - ib, 0, nx)
                else:
                    @pl.when(B < _NBIG - 1)
                    def _():
                        _drain_idx(1 - ib, goff)
                        _fire_gather(1 - ib, 0, nx)
            _drain_gather(m)
            _compute(ib, k, m)

        pltpu.sync_copy(ed_h.at[pl.ds(row0, _BIGC)], edata.at[0])
        if goff is not None:
            for r in range(_BIGC):
                for j in range(_LANES // _DIM):
                    sl = pl.ds(j * _DIM, _DIM)
                    edata[0, r, 0, sl] = edata[0, r, 0, sl] + goff
        _fire_idx(1, row0 + _BIGC)
        _fire_gather(0, 0, 0)
        for k in range(_BIGC):
            _chunk(0, k, 0)
        start = 1
        if (_NBIG - start) % 2 != 0:
            for k in range(_BIGC):
                _chunk(1, k, 1)
            start = 2

        def _pair(t, _):
            for bb in range(2):
                B = 2 * t + start + bb
                ib = (start + bb) % 2
                for k in range(_BIGC):
                    _chunk(ib, k, B)
            return 0

        lax.fori_loop(0, (_NBIG - start) // 2, _pair, 0)
        for m in (1, 2, 3):
            _drain_scatters(m)

    lax.fori_loop(0, _LANES, _zrow, 0)
    _zero_acc(acc_a)
    _zero_acc(acc_b)
    plsc.subcore_barrier()

    _pipeline(1)
    plsc.subcore_barrier()

    pltpu.sync_copy(acc_a.at[pl.ds(nbase, _NPT)],
                    out1_h.at[pl.ds(cc * _NP + nbase, _NPT)])
    lax.fori_loop(0, _LANES, _zrow, 0)
    _zero_acc(acc_a)
    plsc.subcore_barrier()

    _pipeline(2)
    plsc.subcore_barrier()

    pltpu.sync_copy(acc_a.at[pl.ds(nbase, _NPT)],
                    out2_h.at[cc, 0, pl.ds(nbase, _NPT)])
    pltpu.sync_copy(acc_b.at[pl.ds(nbase, _NPT)],
                    out2_h.at[cc, 1, pl.ds(nbase, _NPT)])


def _msgpass(table, edata):
    mesh = plsc.VectorSubcoreMesh(
        core_axis_name="c", subcore_axis_name="s",
        num_cores=_NC, num_subcores=_NS,
    )
    fn = functools.partial(
        pl.kernel,
        out_type=(
            jax.ShapeDtypeStruct((_NC * _NP, _DIM), jnp.float32),
            jax.ShapeDtypeStruct((_NC, 2, _NP, _DIM), jnp.float32),
        ),
        mesh=mesh,
        compiler_params=pltpu.CompilerParams(
            use_tc_tiling_on_sc=False, needs_layout_passes=False),
        scratch_types=[
            pltpu.VMEM_SHARED((_NP, _DIM), jnp.float32),
            pltpu.VMEM_SHARED((_NP, _DIM), jnp.float32),
            pltpu.VMEM((2, _BIGC, 4, _LANES), jnp.int32),
            pltpu.VMEM((_NBANK, 2 * _LANES, _DIM), jnp.float32),
            [pltpu.SemaphoreType.DMA] * _NBANK,
            [pltpu.SemaphoreType.DMA] * _NBANK,
            pltpu.SemaphoreType.DMA,
        ],
    )(_msgpass_body)
    return fn(table, edata)


def _mlp_body(h_ref, c0_ref, c1_ref, p2_ref,
              w1_ref, b1_ref, w2_ref, b2_ref, o_ref):
    feat = jnp.concatenate(
        [h_ref[...], c0_ref[...], c1_ref[...],
         p2_ref[0, 0], p2_ref[1, 0], p2_ref[0, 1], p2_ref[1, 1]], axis=1)
    hid = jnp.dot(feat, w1_ref[...], preferred_element_type=jnp.float32)
    hid = jnp.maximum(hid + b1_ref[...][None, :], 0.0)
    o_ref[...] = (
        jnp.dot(hid, w2_ref[...], preferred_element_type=jnp.float32)
        + b2_ref[...][None, :])


def _mlp(h, c0, c1, p2, W1, b1, W2, b2):
    B = 2000
    part = pl.BlockSpec((_NC, 2, B, _DIM), lambda i: (0, 0, i, 0))
    nd = pl.BlockSpec((B, _DIM), lambda i: (i, 0))
    return pl.pallas_call(
        _mlp_body,
        grid=(_N // B,),
        in_specs=[
            nd, nd, nd, part,
            pl.BlockSpec(W1.shape, lambda i: (0, 0)),
            pl.BlockSpec(b1.shape, lambda i: (0,)),
            pl.BlockSpec(W2.shape, lambda i: (0, 0)),
            pl.BlockSpec(b2.shape, lambda i: (0,)),
        ],
        out_specs=pl.BlockSpec((B, 32), lambda i: (i, 0)),
        out_shape=jax.ShapeDtypeStruct((_N, 32), jnp.float32),
    )(h, c0, c1, p2, W1, b1, W2, b2)


def kernel(x, gauges, kernel_vals, edge_index, n_id, W1, b1, W2, b2):
    src = edge_index[0].astype(jnp.int32)
    dst = edge_index[1].astype(jnp.int32)
    kv0 = lax.bitcast_convert_type(
        kernel_vals[0].astype(jnp.float32), jnp.int32)
    kv1 = lax.bitcast_convert_type(
        kernel_vals[1].astype(jnp.float32), jnp.int32)
    pad = _EP - _E

    def _pad2d(a):
        return jnp.pad(a, (0, pad)).reshape(_R, _LANES)

    edata = jnp.stack([_pad2d(src), _pad2d(dst), _pad2d(kv0), _pad2d(kv1)],
                      axis=1)

    h = _prep(x, gauges)
    out1, p2 = _msgpass(h, edata)
    c0 = lax.slice(out1, (0, 0), (_N, _DIM))
    c1 = lax.slice(out1, (_NP, 0), (_NP + _N, _DIM))
    return _mlp(h, c0, c1, p2, W1, b1, W2, b2)


# speedup vs baseline: 1.0175x; 1.0175x over previous
import functools

import jax
import jax.numpy as jnp
from jax import lax
from jax.experimental import pallas as pl
from jax.experimental.pallas import tpu as pltpu
from jax.experimental.pallas import tpu_sc as plsc

_N = 50000
_DIM = 16
_E = 1600000
_NC = 2
_NS = 16
_LANES = 128
_BIGC = 8
_NBANK = 4
_RPW = 784
_R = _NS * _RPW
_EP = _R * _LANES
_NBIG = _RPW // _BIGC
_NP = 50048
_NPT = _NP // _NS


def _prep_body(x_ref, g_ref, e_ref, r_ref, h_ref):
    xb = jnp.dot(x_ref[...], e_ref[...], preferred_element_type=jnp.float32)
    prod = xb * g_ref[...]
    h = jnp.dot(prod, r_ref[...], preferred_element_type=jnp.float32)
    nrm = jnp.maximum(jnp.sqrt(jnp.sum(h * h, axis=1, keepdims=True)), 1e-12)
    h_ref[...] = h / nrm


def _prep(x, gauges):
    B = 2000
    g2 = gauges.reshape(_N, _DIM * _DIM)
    em = jnp.kron(jnp.eye(_DIM, dtype=jnp.float32),
                  jnp.ones((1, _DIM), dtype=jnp.float32))
    rm = jnp.kron(jnp.ones((_DIM, 1), dtype=jnp.float32),
                  jnp.eye(_DIM, dtype=jnp.float32))
    return pl.pallas_call(
        _prep_body,
        grid=(_N // B,),
        in_specs=[
            pl.BlockSpec((B, _DIM), lambda i: (i, 0)),
            pl.BlockSpec((B, _DIM * _DIM), lambda i: (i, 0)),
            pl.BlockSpec(em.shape, lambda i: (0, 0)),
            pl.BlockSpec(rm.shape, lambda i: (0, 0)),
        ],
        out_specs=pl.BlockSpec((B, _DIM), lambda i: (i, 0)),
        out_shape=jax.ShapeDtypeStruct((_N, _DIM), jnp.float32),
    )(x, g2, em, rm)


def _msgpass_body(table_h, src_h, dst_h, kv0_h, kv1_h, out1_h, out2_h,
                  acc_a, acc_b, sb, db, k0b, k1b, rm, gsems, ssems, isem):
    cc = lax.axis_index("c")
    s = lax.axis_index("s")
    nbase = s * _NPT
    row0 = s * _RPW

    def _zrow(i, _):
        rm[0, i] = jnp.zeros((_DIM,), jnp.float32)
        return 0

    def _zero_acc(acc):
        def _zcp(i, _):
            pltpu.sync_copy(rm.at[0, pl.ds(0, _LANES)],
                            acc.at[pl.ds(nbase + i * _LANES, _LANES)])
            return 0

        nfull = _NPT // _LANES
        lax.fori_loop(0, nfull, _zcp, 0)
        rem = _NPT - nfull * _LANES
        pltpu.sync_copy(rm.at[0, pl.ds(0, rem)],
                        acc.at[pl.ds(nbase + nfull * _LANES, rem)])

    idx_pairs = ((src_h, sb), (dst_h, db), (kv0_h, k0b), (kv1_h, k1b))

    def _fire_idx(ib, r0):
        for hbm, buf in idx_pairs:
            pltpu.async_copy(hbm.at[pl.ds(r0, _BIGC)], buf.at[ib], isem)

    def _drain_idx(ib, goff):
        for hbm, buf in idx_pairs:
            pltpu.make_async_copy(hbm.at[pl.ds(0, _BIGC)], buf.at[ib],
                                  isem).wait()
        if goff is not None:
            for r in range(_BIGC):
                for j in range(_LANES // _DIM):
                    sl = pl.ds(j * _DIM, _DIM)
                    sb[ib, r, sl] = sb[ib, r, sl] + goff

    def _drain_gather(m):
        pltpu.make_async_copy(table_h.at[pl.ds(0, _LANES)],
                              rm.at[m, pl.ds(0, _LANES)], gsems[m]).wait()

    def _pipeline(phase):
        goff = None if phase == 1 else cc * _NP

        def _fire_gather(ib, k, m):
            tbl = table_h if phase == 1 else out1_h
            pltpu.async_copy(tbl.at[sb.at[ib, k]],
                             rm.at[m, pl.ds(0, _LANES)], gsems[m])

        def _drain_scatters(m):
            if phase == 1:
                pltpu.make_async_copy(table_h.at[pl.ds(0, _LANES)],
                                      rm.at[m, pl.ds(0, _LANES)],
                                      ssems[m]).wait()
            else:
                pltpu.make_async_copy(table_h.at[pl.ds(0, 2 * _LANES)],
                                      rm.at[m], ssems[m]).wait()

        def _compute(ib, k, m):
            if phase == 1:
                def _blk(b, _):
                    k0 = plsc.bitcast(
                        k0b[ib, k, pl.ds(b * _DIM, _DIM)], jnp.float32)
                    k1 = plsc.bitcast(
                        k1b[ib, k, pl.ds(b * _DIM, _DIM)], jnp.float32)
                    kv = jnp.where(cc == 0, k0, k1)
                    for ee in range(_DIM):
                        e = b * _DIM + ee
                        rm[m, e] = rm[m, e] * kv[ee]
                    return 0

                lax.fori_loop(0, _LANES // _DIM, _blk, 0)
                pltpu.async_copy(rm.at[m, pl.ds(0, _LANES)],
                                 acc_a.at[db.at[ib, k]], ssems[m],
                                 add=True)
            else:
                def _blk(b, _):
                    k0 = plsc.bitcast(
                        k0b[ib, k, pl.ds(b * _DIM, _DIM)], jnp.float32)
                    k1 = plsc.bitcast(
                        k1b[ib, k, pl.ds(b * _DIM, _DIM)], jnp.float32)
                    for ee in range(_DIM):
                        e = b * _DIM + ee
                        v = rm[m, e]
                        rm[m, _LANES + e] = v * k1[ee]
                        rm[m, e] = v * k0[ee]
                    return 0

                lax.fori_loop(0, _LANES // _DIM, _blk, 0)
                pltpu.async_copy(rm.at[m, pl.ds(0, _LANES)],
                                 acc_a.at[db.at[ib, k]], ssems[m],
                                 add=True)
                pltpu.async_copy(rm.at[m, pl.ds(_LANES, _LANES)],
                                 acc_b.at[db.at[ib, k]], ssems[m],
                                 add=True)

        def _chunk(ib, k, B):
            static = isinstance(B, int)
            m = k % _NBANK
            nx = (k + 1) % _NBANK
            if not (static and B == 0 and k < 3):
                _drain_scatters(nx)
            if k == 3 and not (static and B == 0):
                if static:
                    if B < _NBIG - 1:
                        _fire_idx(1 - ib, row0 + (B + 1) * _BIGC)
                else:
                    @pl.when(B < _NBIG - 1)
                    def _():
                        _fire_idx(1 - ib, row0 + (B + 1) * _BIGC)
            if k < _BIGC - 1:
                _fire_gather(ib, k + 1, nx)
            else:
                if static:
                    if B < _NBIG - 1:
                        _drain_idx(1 - ib, goff)
                        _fire_gather(1 - ib, 0, nx)
                else:
                    @pl.when(B < _NBIG - 1)
                    def _():
                        _drain_idx(1 - ib, goff)
                        _fire_gather(1 - ib, 0, nx)
            _drain_gather(m)
            _compute(ib, k, m)

        for hbm, buf in idx_pairs:
            pltpu.sync_copy(hbm.at[pl.ds(row0, _BIGC)], buf.at[0])
        if goff is not None:
            for r in range(_BIGC):
                for j in range(_LANES // _DIM):
                    sl = pl.ds(j * _DIM, _DIM)
                    sb[0, r, sl] = sb[0, r, sl] + goff
        _fire_idx(1, row0 + _BIGC)
        _fire_gather(0, 0, 0)
        for k in range(_BIGC):
            _chunk(0, k, 0)
        start = 1
        if (_NBIG - start) % 2 != 0:
            for k in range(_BIGC):
                _chunk(1, k, 1)
            start = 2

        def _pair(t, _):
            for bb in range(2):
                B = 2 * t + start + bb
                ib = (start + bb) % 2
                for k in range(_BIGC):
                    _chunk(ib, k, B)
            return 0

        lax.fori_loop(0, (_NBIG - start) // 2, _pair, 0)
        for m in (1, 2, 3):
            _drain_scatters(m)

    lax.fori_loop(0, _LANES, _zrow, 0)
    _zero_acc(acc_a)
    _zero_acc(acc_b)
    plsc.subcore_barrier()

    _pipeline(1)
    plsc.subcore_barrier()

    pltpu.sync_copy(acc_a.at[pl.ds(nbase, _NPT)],
                    out1_h.at[pl.ds(cc * _NP + nbase, _NPT)])
    lax.fori_loop(0, _LANES, _zrow, 0)
    _zero_acc(acc_a)
    plsc.subcore_barrier()

    _pipeline(2)
    plsc.subcore_barrier()

    pltpu.sync_copy(acc_a.at[pl.ds(nbase, _NPT)],
                    out2_h.at[cc, 0, pl.ds(nbase, _NPT)])
    pltpu.sync_copy(acc_b.at[pl.ds(nbase, _NPT)],
                    out2_h.at[cc, 1, pl.ds(nbase, _NPT)])


def _msgpass(table, srcp, dstp, kv0p, kv1p):
    mesh = plsc.VectorSubcoreMesh(
        core_axis_name="c", subcore_axis_name="s",
        num_cores=_NC, num_subcores=_NS,
    )
    fn = functools.partial(
        pl.kernel,
        out_type=(
            jax.ShapeDtypeStruct((_NC * _NP, _DIM), jnp.float32),
            jax.ShapeDtypeStruct((_NC, 2, _NP, _DIM), jnp.float32),
        ),
        mesh=mesh,
        compiler_params=pltpu.CompilerParams(
            use_tc_tiling_on_sc=False, needs_layout_passes=False),
        scratch_types=[
            pltpu.VMEM_SHARED((_NP, _DIM), jnp.float32),
            pltpu.VMEM_SHARED((_NP, _DIM), jnp.float32),
            pltpu.VMEM((2, _BIGC, _LANES), jnp.int32),
            pltpu.VMEM((2, _BIGC, _LANES), jnp.int32),
            pltpu.VMEM((2, _BIGC, _LANES), jnp.int32),
            pltpu.VMEM((2, _BIGC, _LANES), jnp.int32),
            pltpu.VMEM((_NBANK, 2 * _LANES, _DIM), jnp.float32),
            [pltpu.SemaphoreType.DMA] * _NBANK,
            [pltpu.SemaphoreType.DMA] * _NBANK,
            pltpu.SemaphoreType.DMA,
        ],
    )(_msgpass_body)
    return fn(table, srcp, dstp, kv0p, kv1p)


def _mlp_body(h_ref, c0_ref, c1_ref, p2_ref,
              w1_ref, b1_ref, w2_ref, b2_ref, o_ref):
    feat = jnp.concatenate(
        [h_ref[...], c0_ref[...], c1_ref[...],
         p2_ref[0, 0], p2_ref[1, 0], p2_ref[0, 1], p2_ref[1, 1]], axis=1)
    hid = jnp.dot(feat, w1_ref[...], preferred_element_type=jnp.float32)
    hid = jnp.maximum(hid + b1_ref[...][None, :], 0.0)
    o_ref[...] = (
        jnp.dot(hid, w2_ref[...], preferred_element_type=jnp.float32)
        + b2_ref[...][None, :])


def _mlp(h, c0, c1, p2, W1, b1, W2, b2):
    B = 2000
    part = pl.BlockSpec((_NC, 2, B, _DIM), lambda i: (0, 0, i, 0))
    nd = pl.BlockSpec((B, _DIM), lambda i: (i, 0))
    return pl.pallas_call(
        _mlp_body,
        grid=(_N // B,),
        in_specs=[
            nd, nd, nd, part,
            pl.BlockSpec(W1.shape, lambda i: (0, 0)),
            pl.BlockSpec(b1.shape, lambda i: (0,)),
            pl.BlockSpec(W2.shape, lambda i: (0, 0)),
            pl.BlockSpec(b2.shape, lambda i: (0,)),
        ],
        out_specs=pl.BlockSpec((B, 32), lambda i: (i, 0)),
        out_shape=jax.ShapeDtypeStruct((_N, 32), jnp.float32),
    )(h, c0, c1, p2, W1, b1, W2, b2)


def kernel(x, gauges, kernel_vals, edge_index, n_id, W1, b1, W2, b2):
    src = edge_index[0].astype(jnp.int32)
    dst = edge_index[1].astype(jnp.int32)
    kv0 = lax.bitcast_convert_type(
        kernel_vals[0].astype(jnp.float32), jnp.int32)
    kv1 = lax.bitcast_convert_type(
        kernel_vals[1].astype(jnp.float32), jnp.int32)
    pad = _EP - _E

    def _pad2d(a):
        return jnp.pad(a, (0, pad)).reshape(_R, _LANES)

    h = _prep(x, gauges)
    out1, p2 = _msgpass(h, _pad2d(src), _pad2d(dst), _pad2d(kv0), _pad2d(kv1))
    c0 = lax.slice(out1, (0, 0), (_N, _DIM))
    c1 = lax.slice(out1, (_NP, 0), (_NP + _N, _DIM))
    return _mlp(h, c0, c1, p2, W1, b1, W2, b2)


# speedup vs baseline: 1.1150x; 1.0958x over previous
import functools

import jax
import jax.numpy as jnp
from jax import lax
from jax.experimental import pallas as pl
from jax.experimental.pallas import tpu as pltpu
from jax.experimental.pallas import tpu_sc as plsc

_N = 50000
_DIM = 16
_E = 1600000
_NC = 2
_NS = 16
_LANES = 128
_BIGC = 8
_NBANK = 4
_RPW = 784
_ER = _E // _LANES
_NBIG = _RPW // _BIGC
_NP = 50048
_NPT = _NP // _NS


def _prep_body(x_ref, g_ref, e_ref, r_ref, h_ref):
    xb = jnp.dot(x_ref[...], e_ref[...], preferred_element_type=jnp.float32)
    prod = xb * g_ref[...]
    h = jnp.dot(prod, r_ref[...], preferred_element_type=jnp.float32)
    nrm = jnp.maximum(jnp.sqrt(jnp.sum(h * h, axis=1, keepdims=True)), 1e-12)
    h_ref[...] = h / nrm


def _prep(x, gauges):
    B = 2000
    g2 = gauges.reshape(_N, _DIM * _DIM)
    em = jnp.kron(jnp.eye(_DIM, dtype=jnp.float32),
                  jnp.ones((1, _DIM), dtype=jnp.float32))
    rm = jnp.kron(jnp.ones((_DIM, 1), dtype=jnp.float32),
                  jnp.eye(_DIM, dtype=jnp.float32))
    return pl.pallas_call(
        _prep_body,
        grid=(_N // B,),
        in_specs=[
            pl.BlockSpec((B, _DIM), lambda i: (i, 0)),
            pl.BlockSpec((B, _DIM * _DIM), lambda i: (i, 0)),
            pl.BlockSpec(em.shape, lambda i: (0, 0)),
            pl.BlockSpec(rm.shape, lambda i: (0, 0)),
        ],
        out_specs=pl.BlockSpec((B, _DIM), lambda i: (i, 0)),
        out_shape=jax.ShapeDtypeStruct((_N, _DIM), jnp.float32),
    )(x, g2, em, rm)


def _msgpass_body(table_h, ei_h, kv_h, out1_h, out2_h,
                  acc_a, acc_b, sb, db, k0b, k1b, rm, gsems, ssems, isem):
    cc = lax.axis_index("c")
    s = lax.axis_index("s")
    nbase = s * _NPT
    row0 = s * _RPW

    def _zrow(i, _):
        rm[0, i] = jnp.zeros((_DIM,), jnp.float32)
        return 0

    def _zero_acc(acc):
        def _zcp(i, _):
            pltpu.sync_copy(rm.at[0, pl.ds(0, _LANES)],
                            acc.at[pl.ds(nbase + i * _LANES, _LANES)])
            return 0

        nfull = _NPT // _LANES
        lax.fori_loop(0, nfull, _zcp, 0)
        rem = _NPT - nfull * _LANES
        pltpu.sync_copy(rm.at[0, pl.ds(0, rem)],
                        acc.at[pl.ds(nbase + nfull * _LANES, rem)])

    idx_quads = ((ei_h, 0, sb), (ei_h, 1, db), (kv_h, 0, k0b), (kv_h, 1, k1b))

    def _fire_idx(ib, r0):
        r0c = jnp.minimum(r0, _ER - _BIGC)
        for hbm, a, buf in idx_quads:
            pltpu.async_copy(hbm.at[a, pl.ds(r0c, _BIGC)], buf.at[ib], isem)

    def _drain_idx(ib, goff):
        for hbm, a, buf in idx_quads:
            pltpu.make_async_copy(hbm.at[a, pl.ds(0, _BIGC)], buf.at[ib],
                                  isem).wait()
        if goff is not None:
            for r in range(_BIGC):
                for j in range(_LANES // _DIM):
                    sl = pl.ds(j * _DIM, _DIM)
                    sb[ib, r, sl] = sb[ib, r, sl] + goff

    def _drain_gather(m):
        pltpu.make_async_copy(table_h.at[pl.ds(0, _LANES)],
                              rm.at[m, pl.ds(0, _LANES)], gsems[m]).wait()

    def _pipeline(phase):
        goff = None if phase == 1 else cc * _NP

        def _fire_gather(ib, k, m):
            tbl = table_h if phase == 1 else out1_h
            pltpu.async_copy(tbl.at[sb.at[ib, k]],
                             rm.at[m, pl.ds(0, _LANES)], gsems[m])

        def _drain_scatters(m):
            if phase == 1:
                pltpu.make_async_copy(table_h.at[pl.ds(0, _LANES)],
                                      rm.at[m, pl.ds(0, _LANES)],
                                      ssems[m]).wait()
            else:
                pltpu.make_async_copy(table_h.at[pl.ds(0, 2 * _LANES)],
                                      rm.at[m], ssems[m]).wait()

        def _compute(ib, k, m, B):
            r0 = row0 + B * _BIGC
            delta = r0 - jnp.minimum(r0, _ER - _BIGC)
            fm = jnp.where(delta <= k, 1.0, 0.0)
            if phase == 1:
                def _blk(b, _):
                    k0 = k0b[ib, k, pl.ds(b * _DIM, _DIM)]
                    k1 = k1b[ib, k, pl.ds(b * _DIM, _DIM)]
                    kv = jnp.where(cc == 0, k0, k1) * fm
                    for ee in range(_DIM):
                        e = b * _DIM + ee
                        rm[m, e] = rm[m, e] * kv[ee]
                    return 0

                lax.fori_loop(0, _LANES // _DIM, _blk, 0)
                pltpu.async_copy(rm.at[m, pl.ds(0, _LANES)],
                                 acc_a.at[db.at[ib, k]], ssems[m],
                                 add=True)
            else:
                def _blk(b, _):
                    k0 = k0b[ib, k, pl.ds(b * _DIM, _DIM)] * fm
                    k1 = k1b[ib, k, pl.ds(b * _DIM, _DIM)] * fm
                    for ee in range(_DIM):
                        e = b * _DIM + ee
                        v = rm[m, e]
                        rm[m, _LANES + e] = v * k1[ee]
                        rm[m, e] = v * k0[ee]
                    return 0

                lax.fori_loop(0, _LANES // _DIM, _blk, 0)
                pltpu.async_copy(rm.at[m, pl.ds(0, _LANES)],
                                 acc_a.at[db.at[ib, k]], ssems[m],
                                 add=True)
                pltpu.async_copy(rm.at[m, pl.ds(_LANES, _LANES)],
                                 acc_b.at[db.at[ib, k]], ssems[m],
                                 add=True)

        def _chunk(ib, k, B):
            static = isinstance(B, int)
            m = k % _NBANK
            nx = (k + 1) % _NBANK
            if not (static and B == 0 and k < 3):
                _drain_scatters(nx)
            if k == 3 and not (static and B == 0):
                if static:
                    if B < _NBIG - 1:
                        _fire_idx(1 - ib, row0 + (B + 1) * _BIGC)
                else:
                    @pl.when(B < _NBIG - 1)
                    def _():
                        _fire_idx(1 - ib, row0 + (B + 1) * _BIGC)
            if k < _BIGC - 1:
                _fire_gather(ib, k + 1, nx)
            else:
                if static:
                    if B < _NBIG - 1:
                        _drain_idx(1 - ib, goff)
                        _fire_gather(1 - ib, 0, nx)
                else:
                    @pl.when(B < _NBIG - 1)
                    def _():
                        _drain_idx(1 - ib, goff)
                        _fire_gather(1 - ib, 0, nx)
            _drain_gather(m)
            _compute(ib, k, m, B)

        for hbm, a, buf in idx_quads:
            pltpu.sync_copy(hbm.at[a, pl.ds(row0, _BIGC)], buf.at[0])
        if goff is not None:
            for r in range(_BIGC):
                for j in range(_LANES // _DIM):
                    sl = pl.ds(j * _DIM, _DIM)
                    sb[0, r, sl] = sb[0, r, sl] + goff
        _fire_idx(1, row0 + _BIGC)
        _fire_gather(0, 0, 0)
        for k in range(_BIGC):
            _chunk(0, k, 0)
        start = 1
        if (_NBIG - start) % 2 != 0:
            for k in range(_BIGC):
                _chunk(1, k, 1)
            start = 2

        def _pair(t, _):
            for bb in range(2):
                B = 2 * t + start + bb
                ib = (start + bb) % 2
                for k in range(_BIGC):
                    _chunk(ib, k, B)
            return 0

        lax.fori_loop(0, (_NBIG - start) // 2, _pair, 0)
        for m in (1, 2, 3):
            _drain_scatters(m)

    lax.fori_loop(0, _LANES, _zrow, 0)
    _zero_acc(acc_a)
    _zero_acc(acc_b)
    plsc.subcore_barrier()

    _pipeline(1)
    plsc.subcore_barrier()

    pltpu.sync_copy(acc_a.at[pl.ds(nbase, _NPT)],
                    out1_h.at[pl.ds(cc * _NP + nbase, _NPT)])
    lax.fori_loop(0, _LANES, _zrow, 0)
    _zero_acc(acc_a)
    plsc.subcore_barrier()

    _pipeline(2)
    plsc.subcore_barrier()

    pltpu.sync_copy(acc_a.at[pl.ds(nbase, _NPT)],
                    out2_h.at[cc, 0, pl.ds(nbase, _NPT)])
    pltpu.sync_copy(acc_b.at[pl.ds(nbase, _NPT)],
                    out2_h.at[cc, 1, pl.ds(nbase, _NPT)])


def _msgpass(table, ei3, kv3):
    mesh = plsc.VectorSubcoreMesh(
        core_axis_name="c", subcore_axis_name="s",
        num_cores=_NC, num_subcores=_NS,
    )
    fn = functools.partial(
        pl.kernel,
        out_type=(
            jax.ShapeDtypeStruct((_NC * _NP, _DIM), jnp.float32),
            jax.ShapeDtypeStruct((_NC, 2, _NP, _DIM), jnp.float32),
        ),
        mesh=mesh,
        compiler_params=pltpu.CompilerParams(
            use_tc_tiling_on_sc=False, needs_layout_passes=False),
        scratch_types=[
            pltpu.VMEM_SHARED((_NP, _DIM), jnp.float32),
            pltpu.VMEM_SHARED((_NP, _DIM), jnp.float32),
            pltpu.VMEM((2, _BIGC, _LANES), jnp.int32),
            pltpu.VMEM((2, _BIGC, _LANES), jnp.int32),
            pltpu.VMEM((2, _BIGC, _LANES), jnp.float32),
            pltpu.VMEM((2, _BIGC, _LANES), jnp.float32),
            pltpu.VMEM((_NBANK, 2 * _LANES, _DIM), jnp.float32),
            [pltpu.SemaphoreType.DMA] * _NBANK,
            [pltpu.SemaphoreType.DMA] * _NBANK,
            pltpu.SemaphoreType.DMA,
        ],
    )(_msgpass_body)
    return fn(table, ei3, kv3)


def _mlp_body(h_ref, c0_ref, c1_ref, p2_ref,
              w1_ref, b1_ref, w2_ref, b2_ref, o_ref):
    feat = jnp.concatenate(
        [h_ref[...], c0_ref[...], c1_ref[...],
         p2_ref[0, 0], p2_ref[1, 0], p2_ref[0, 1], p2_ref[1, 1]], axis=1)
    hid = jnp.dot(feat, w1_ref[...], preferred_element_type=jnp.float32)
    hid = jnp.maximum(hid + b1_ref[...][None, :], 0.0)
    o_ref[...] = (
        jnp.dot(hid, w2_ref[...], preferred_element_type=jnp.float32)
        + b2_ref[...][None, :])


def _mlp(h, c0, c1, p2, W1, b1, W2, b2):
    B = 2000
    part = pl.BlockSpec((_NC, 2, B, _DIM), lambda i: (0, 0, i, 0))
    nd = pl.BlockSpec((B, _DIM), lambda i: (i, 0))
    return pl.pallas_call(
        _mlp_body,
        grid=(_N // B,),
        in_specs=[
            nd, nd, nd, part,
            pl.BlockSpec(W1.shape, lambda i: (0, 0)),
            pl.BlockSpec(b1.shape, lambda i: (0,)),
            pl.BlockSpec(W2.shape, lambda i: (0, 0)),
            pl.BlockSpec(b2.shape, lambda i: (0,)),
        ],
        out_specs=pl.BlockSpec((B, 32), lambda i: (i, 0)),
        out_shape=jax.ShapeDtypeStruct((_N, 32), jnp.float32),
    )(h, c0, c1, p2, W1, b1, W2, b2)


def kernel(x, gauges, kernel_vals, edge_index, n_id, W1, b1, W2, b2):
    ei3 = edge_index.astype(jnp.int32).reshape(2, _ER, _LANES)
    kv3 = kernel_vals.astype(jnp.float32).reshape(2, _ER, _LANES)

    h = _prep(x, gauges)
    out1, p2 = _msgpass(h, ei3, kv3)
    c0 = lax.slice(out1, (0, 0), (_N, _DIM))
    c1 = lax.slice(out1, (_NP, 0), (_NP + _N, _DIM))
    return _mlp(h, c0, c1, p2, W1, b1, W2, b2)


# speedup vs baseline: 1.1420x; 1.0241x over previous
import functools

import jax
import jax.numpy as jnp
from jax import lax
from jax.experimental import pallas as pl
from jax.experimental.pallas import tpu as pltpu
from jax.experimental.pallas import tpu_sc as plsc

_N = 50000
_DIM = 16
_E = 1600000
_NC = 2
_NS = 16
_LANES = 128
_BIGC = 8
_NBANK = 4
_RPW = 784
_ER = _E // _LANES
_NBIG = _RPW // _BIGC
_NP = 50048
_NPT = _NP // _NS


def _prep_body(x_ref, g_ref, e_ref, r_ref, h_ref):
    xb = jnp.dot(x_ref[...], e_ref[...], preferred_element_type=jnp.float32)
    prod = xb * g_ref[...]
    h = jnp.dot(prod, r_ref[...], preferred_element_type=jnp.float32)
    nrm = jnp.maximum(jnp.sqrt(jnp.sum(h * h, axis=1, keepdims=True)), 1e-12)
    h_ref[...] = h / nrm


def _prep(x, gauges):
    B = 2000
    g2 = gauges.reshape(_N, _DIM * _DIM)
    em = jnp.kron(jnp.eye(_DIM, dtype=jnp.float32),
                  jnp.ones((1, _DIM), dtype=jnp.float32))
    rm = jnp.kron(jnp.ones((_DIM, 1), dtype=jnp.float32),
                  jnp.eye(_DIM, dtype=jnp.float32))
    return pl.pallas_call(
        _prep_body,
        grid=(_N // B,),
        in_specs=[
            pl.BlockSpec((B, _DIM), lambda i: (i, 0)),
            pl.BlockSpec((B, _DIM * _DIM), lambda i: (i, 0)),
            pl.BlockSpec(em.shape, lambda i: (0, 0)),
            pl.BlockSpec(rm.shape, lambda i: (0, 0)),
        ],
        out_specs=pl.BlockSpec((B, _DIM), lambda i: (i, 0)),
        out_shape=jax.ShapeDtypeStruct((_N, _DIM), jnp.float32),
    )(x, g2, em, rm)


def _msgpass_body(table_h, ei_h, kv_h, out1_h, out1b_h, out2_h,
                  acc_a, acc_b, sb, db, k0b, k1b, rm, gsems, ssems, isem):
    cc = lax.axis_index("c")
    s = lax.axis_index("s")
    nbase = s * _NPT
    row0 = s * _RPW

    def _zrow(i, _):
        rm[0, i] = jnp.zeros((_DIM,), jnp.float32)
        return 0

    def _zero_acc(acc):
        def _zcp(i, _):
            pltpu.sync_copy(rm.at[0, pl.ds(0, _LANES)],
                            acc.at[pl.ds(nbase + i * _LANES, _LANES)])
            return 0

        nfull = _NPT // _LANES
        lax.fori_loop(0, nfull, _zcp, 0)
        rem = _NPT - nfull * _LANES
        pltpu.sync_copy(rm.at[0, pl.ds(0, rem)],
                        acc.at[pl.ds(nbase + nfull * _LANES, rem)])

    idx_quads = ((ei_h, 0, sb), (ei_h, 1, db), (kv_h, 0, k0b), (kv_h, 1, k1b))

    def _fire_idx(ib, r0):
        r0c = jnp.minimum(r0, _ER - _BIGC)
        for hbm, a, buf in idx_quads:
            pltpu.async_copy(hbm.at[a, pl.ds(r0c, _BIGC)], buf.at[ib], isem)

    def _drain_idx(ib, goff):
        for hbm, a, buf in idx_quads:
            pltpu.make_async_copy(hbm.at[a, pl.ds(0, _BIGC)], buf.at[ib],
                                  isem).wait()
        if goff is not None:
            for r in range(_BIGC):
                for j in range(_LANES // _DIM):
                    sl = pl.ds(j * _DIM, _DIM)
                    sb[ib, r, sl] = sb[ib, r, sl] + goff

    def _drain_gather(m):
        pltpu.make_async_copy(table_h.at[pl.ds(0, _LANES)],
                              rm.at[m, pl.ds(0, _LANES)], gsems[m]).wait()

    def _pipeline(phase):
        goff = None if phase == 1 else cc * _NP

        def _fire_gather(ib, k, m):
            tbl = table_h if phase == 1 else out1_h
            pltpu.async_copy(tbl.at[sb.at[ib, k]],
                             rm.at[m, pl.ds(0, _LANES)], gsems[m])

        def _drain_scatters(m):
            if phase == 1:
                pltpu.make_async_copy(table_h.at[pl.ds(0, _LANES)],
                                      rm.at[m, pl.ds(0, _LANES)],
                                      ssems[m]).wait()
            else:
                pltpu.make_async_copy(table_h.at[pl.ds(0, 2 * _LANES)],
                                      rm.at[m], ssems[m]).wait()

        def _compute(ib, k, m, B):
            r0 = row0 + B * _BIGC
            delta = r0 - jnp.minimum(r0, _ER - _BIGC)
            fm = jnp.where(delta <= k, 1.0, 0.0)
            if phase == 1:
                def _blk(b, _):
                    k0 = k0b[ib, k, pl.ds(b * _DIM, _DIM)]
                    k1 = k1b[ib, k, pl.ds(b * _DIM, _DIM)]
                    kv = jnp.where(cc == 0, k0, k1) * fm
                    for ee in range(_DIM):
                        e = b * _DIM + ee
                        rm[m, e] = rm[m, e] * kv[ee]
                    return 0

                lax.fori_loop(0, _LANES // _DIM, _blk, 0)
                pltpu.async_copy(rm.at[m, pl.ds(0, _LANES)],
                                 acc_a.at[db.at[ib, k]], ssems[m],
                                 add=True)
            else:
                def _blk(b, _):
                    k0 = k0b[ib, k, pl.ds(b * _DIM, _DIM)] * fm
                    k1 = k1b[ib, k, pl.ds(b * _DIM, _DIM)] * fm
                    for ee in range(_DIM):
                        e = b * _DIM + ee
                        v = rm[m, e]
                        rm[m, _LANES + e] = v * k1[ee]
                        rm[m, e] = v * k0[ee]
                    return 0

                lax.fori_loop(0, _LANES // _DIM, _blk, 0)
                pltpu.async_copy(rm.at[m, pl.ds(0, _LANES)],
                                 acc_a.at[db.at[ib, k]], ssems[m],
                                 add=True)
                pltpu.async_copy(rm.at[m, pl.ds(_LANES, _LANES)],
                                 acc_b.at[db.at[ib, k]], ssems[m],
                                 add=True)

        def _chunk(ib, k, B):
            static = isinstance(B, int)
            m = k % _NBANK
            nx = (k + 1) % _NBANK
            if not (static and B == 0 and k < 3):
                _drain_scatters(nx)
            if k == 3 and not (static and B == 0):
                if static:
                    if B < _NBIG - 1:
                        _fire_idx(1 - ib, row0 + (B + 1) * _BIGC)
                else:
                    @pl.when(B < _NBIG - 1)
                    def _():
                        _fire_idx(1 - ib, row0 + (B + 1) * _BIGC)
            if k < _BIGC - 1:
                _fire_gather(ib, k + 1, nx)
            else:
                if static:
                    if B < _NBIG - 1:
                        _drain_idx(1 - ib, goff)
                        _fire_gather(1 - ib, 0, nx)
                else:
                    @pl.when(B < _NBIG - 1)
                    def _():
                        _drain_idx(1 - ib, goff)
                        _fire_gather(1 - ib, 0, nx)
            _drain_gather(m)
            _compute(ib, k, m, B)

        for hbm, a, buf in idx_quads:
            pltpu.sync_copy(hbm.at[a, pl.ds(row0, _BIGC)], buf.at[0])
        if goff is not None:
            for r in range(_BIGC):
                for j in range(_LANES // _DIM):
                    sl = pl.ds(j * _DIM, _DIM)
                    sb[0, r, sl] = sb[0, r, sl] + goff
        _fire_idx(1, row0 + _BIGC)
        _fire_gather(0, 0, 0)
        for k in range(_BIGC):
            _chunk(0, k, 0)
        start = 1
        if (_NBIG - start) % 2 != 0:
            for k in range(_BIGC):
                _chunk(1, k, 1)
            start = 2

        def _pair(t, _):
            for bb in range(2):
                B = 2 * t + start + bb
                ib = (start + bb) % 2
                for k in range(_BIGC):
                    _chunk(ib, k, B)
            return 0

        lax.fori_loop(0, (_NBIG - start) // 2, _pair, 0)
        for m in (1, 2, 3):
            _drain_scatters(m)

    lax.fori_loop(0, _LANES, _zrow, 0)
    _zero_acc(acc_a)
    _zero_acc(acc_b)
    plsc.subcore_barrier()

    _pipeline(1)
    plsc.subcore_barrier()

    pltpu.sync_copy(acc_a.at[pl.ds(nbase, _NPT)],
                    out1_h.at[pl.ds(cc * _NP + nbase, _NPT)])
    pltpu.sync_copy(acc_a.at[pl.ds(nbase, _NPT)],
                    out1b_h.at[cc, pl.ds(nbase, _NPT)])
    lax.fori_loop(0, _LANES, _zrow, 0)
    _zero_acc(acc_a)
    plsc.subcore_barrier()

    _pipeline(2)
    plsc.subcore_barrier()

    pltpu.sync_copy(acc_a.at[pl.ds(nbase, _NPT)],
                    out2_h.at[cc, 0, pl.ds(nbase, _NPT)])
    pltpu.sync_copy(acc_b.at[pl.ds(nbase, _NPT)],
                    out2_h.at[cc, 1, pl.ds(nbase, _NPT)])


def _msgpass(table, ei3, kv3):
    mesh = plsc.VectorSubcoreMesh(
        core_axis_name="c", subcore_axis_name="s",
        num_cores=_NC, num_subcores=_NS,
    )
    fn = functools.partial(
        pl.kernel,
        out_type=(
            jax.ShapeDtypeStruct((_NC * _NP, _DIM), jnp.float32),
            jax.ShapeDtypeStruct((_NC, _NP, _DIM), jnp.float32),
            jax.ShapeDtypeStruct((_NC, 2, _NP, _DIM), jnp.float32),
        ),
        mesh=mesh,
        compiler_params=pltpu.CompilerParams(
            use_tc_tiling_on_sc=False, needs_layout_passes=False),
        scratch_types=[
            pltpu.VMEM_SHARED((_NP, _DIM), jnp.float32),
            pltpu.VMEM_SHARED((_NP, _DIM), jnp.float32),
            pltpu.VMEM((2, _BIGC, _LANES), jnp.int32),
            pltpu.VMEM((2, _BIGC, _LANES), jnp.int32),
            pltpu.VMEM((2, _BIGC, _LANES), jnp.float32),
            pltpu.VMEM((2, _BIGC, _LANES), jnp.float32),
            pltpu.VMEM((_NBANK, 2 * _LANES, _DIM), jnp.float32),
            [pltpu.SemaphoreType.DMA] * _NBANK,
            [pltpu.SemaphoreType.DMA] * _NBANK,
            pltpu.SemaphoreType.DMA,
        ],
    )(_msgpass_body)
    return fn(table, ei3, kv3)


def _mlp_body(h_ref, c0_ref, c1_ref, p2_ref,
              w1_ref, b1_ref, w2_ref, b2_ref, o_ref):
    feat = jnp.concatenate(
        [h_ref[...], c0_ref[0], c1_ref[0],
         p2_ref[0, 0], p2_ref[1, 0], p2_ref[0, 1], p2_ref[1, 1]], axis=1)
    hid = jnp.dot(feat, w1_ref[...], preferred_element_type=jnp.float32)
    hid = jnp.maximum(hid + b1_ref[...][None, :], 0.0)
    o_ref[...] = (
        jnp.dot(hid, w2_ref[...], preferred_element_type=jnp.float32)
        + b2_ref[...][None, :])


def _mlp(h, c0c1, p2, W1, b1, W2, b2):
    B = 2000
    part = pl.BlockSpec((_NC, 2, B, _DIM), lambda i: (0, 0, i, 0))
    nd = pl.BlockSpec((B, _DIM), lambda i: (i, 0))
    return pl.pallas_call(
        _mlp_body,
        grid=(_N // B,),
        in_specs=[
            nd,
            pl.BlockSpec((1, B, _DIM), lambda i: (0, i, 0)),
            pl.BlockSpec((1, B, _DIM), lambda i: (1, i, 0)),
            part,
            pl.BlockSpec(W1.shape, lambda i: (0, 0)),
            pl.BlockSpec(b1.shape, lambda i: (0,)),
            pl.BlockSpec(W2.shape, lambda i: (0, 0)),
            pl.BlockSpec(b2.shape, lambda i: (0,)),
        ],
        out_specs=pl.BlockSpec((B, 32), lambda i: (i, 0)),
        out_shape=jax.ShapeDtypeStruct((_N, 32), jnp.float32),
    )(h, c0c1, c0c1, p2, W1, b1, W2, b2)


def kernel(x, gauges, kernel_vals, edge_index, n_id, W1, b1, W2, b2):
    ei3 = edge_index.astype(jnp.int32).reshape(2, _ER, _LANES)
    kv3 = kernel_vals.astype(jnp.float32).reshape(2, _ER, _LANES)

    h = _prep(x, gauges)
    out1, out1b, p2 = _msgpass(h, ei3, kv3)
    return _mlp(h, out1b, p2, W1, b1, W2, b2)
